# baseline (device time: 20740 ns/iter reference)
import jax
import jax.numpy as jnp
from jax import lax
from jax.experimental import pallas as pl
from jax.experimental.pallas import tpu as pltpu

N_DEV = 4
N_LOCAL_EXPERTS = 2
N_EXPERTS = 8


def kernel(x, router_W, route_idx, expert_W, shared_W):
    n_tok, d_model = x.shape
    d_ff = expert_W.shape[-1]

    def body(x_ref, rw_ref, idx_ref, ew_ref, sw_ref, out_ref,
             comm_ref, send_sems, recv_sems):
        my_pos = lax.axis_index("i")
        left = (my_pos - 1) % N_DEV
        right = (my_pos + 1) % N_DEV

        barrier_sem = pltpu.get_barrier_semaphore()
        for nbr in [left, right]:
            pl.semaphore_signal(
                barrier_sem, inc=1,
                device_id=(nbr,), device_id_type=pl.DeviceIdType.MESH,
            )
        pl.semaphore_wait(barrier_sem, 2)

        xf = x_ref[...]
        scores = jnp.dot(xf, rw_ref[...], preferred_element_type=jnp.float32)
        scores = scores - jnp.max(scores, axis=-1, keepdims=True)
        e_s = jnp.exp(scores)
        probs = e_s / jnp.sum(e_s, axis=-1, keepdims=True)

        cols = lax.broadcasted_iota(jnp.int32, (n_tok, N_EXPERTS), 1)
        tok_expert = idx_ref[...]

        xb = xf.astype(jnp.bfloat16)
        acc = jnp.zeros((n_tok, d_ff), dtype=jnp.float32)
        for j in range(N_LOCAL_EXPERTS):
            e = my_pos * N_LOCAL_EXPERTS + j
            gate = jnp.sum(
                jnp.where((cols == e) & (tok_expert == e), probs, 0.0),
                axis=1,
            )
            wj = ew_ref[j].astype(jnp.bfloat16)
            y = jnp.dot(xb, wj, preferred_element_type=jnp.float32)
            acc = acc + gate[:, None] * y

        shared = jnp.dot(xb, sw_ref[...].astype(jnp.bfloat16),
                         preferred_element_type=jnp.float32)

        comm_ref[0] = acc
        out_ref[...] = shared + acc

        for h in range(N_DEV - 1):
            rdma = pltpu.make_async_remote_copy(
                src_ref=comm_ref.at[h],
                dst_ref=comm_ref.at[h + 1],
                send_sem=send_sems.at[h],
                recv_sem=recv_sems.at[h],
                device_id=(right,),
                device_id_type=pl.DeviceIdType.MESH,
            )
            rdma.start()
            rdma.wait()
            out_ref[...] = out_ref[...] + comm_ref[h + 1]

    return pl.pallas_call(
        body,
        out_shape=jax.ShapeDtypeStruct((n_tok, d_ff), jnp.float32),
        in_specs=[pl.BlockSpec(memory_space=pltpu.VMEM)] * 5,
        out_specs=pl.BlockSpec(memory_space=pltpu.VMEM),
        scratch_shapes=[
            pltpu.VMEM((N_DEV, n_tok, d_ff), jnp.float32),
            pltpu.SemaphoreType.DMA((N_DEV - 1,)),
            pltpu.SemaphoreType.DMA((N_DEV - 1,)),
        ],
        compiler_params=pltpu.CompilerParams(collective_id=0),
    )(x, router_W, route_idx, expert_W, shared_W)


# device time: 11835 ns/iter; 1.7524x vs baseline; 1.7524x over previous
import jax
import jax.numpy as jnp
from jax import lax
from jax.experimental import pallas as pl
from jax.experimental.pallas import tpu as pltpu

N_DEV = 4
N_LOCAL_EXPERTS = 2
N_EXPERTS = 8


def kernel(x, router_W, route_idx, expert_W, shared_W):
    n_tok, d_model = x.shape
    d_ff = expert_W.shape[-1]

    def body(x_ref, rw_ref, idx_ref, ew_ref, sw_ref, out_ref,
             comm_ref, send_sems, recv_sems):
        my_pos = lax.axis_index("i")
        peers = [(my_pos + k) % N_DEV for k in (1, 2, 3)]

        barrier_sem = pltpu.get_barrier_semaphore()
        for p in peers:
            pl.semaphore_signal(
                barrier_sem, inc=1,
                device_id=(p,), device_id_type=pl.DeviceIdType.MESH,
            )
        pl.semaphore_wait(barrier_sem, N_DEV - 1)

        xf = x_ref[...]
        scores = jnp.dot(xf, rw_ref[...], preferred_element_type=jnp.float32)
        scores = scores - jnp.max(scores, axis=-1, keepdims=True)
        e_s = jnp.exp(scores)
        probs = e_s / jnp.sum(e_s, axis=-1, keepdims=True)

        cols = lax.broadcasted_iota(jnp.int32, (n_tok, N_EXPERTS), 1)
        tok_expert = idx_ref[...]

        xb = xf.astype(jnp.bfloat16)
        acc = jnp.zeros((n_tok, d_ff), dtype=jnp.float32)
        for j in range(N_LOCAL_EXPERTS):
            e = my_pos * N_LOCAL_EXPERTS + j
            gate = jnp.sum(
                jnp.where((cols == e) & (tok_expert == e), probs, 0.0),
                axis=1,
            )
            wj = ew_ref[j].astype(jnp.bfloat16)
            y = jnp.dot(xb, wj, preferred_element_type=jnp.float32)
            acc = acc + gate[:, None] * y

        comm_ref[my_pos] = acc.astype(jnp.bfloat16)
        sends = []
        for i, p in enumerate(peers):
            rdma = pltpu.make_async_remote_copy(
                src_ref=comm_ref.at[my_pos],
                dst_ref=comm_ref.at[my_pos],
                send_sem=send_sems.at[i],
                recv_sem=recv_sems.at[my_pos],
                device_id=(p,),
                device_id_type=pl.DeviceIdType.MESH,
            )
            rdma.start()
            sends.append(rdma)

        shared = jnp.dot(xb, sw_ref[...].astype(jnp.bfloat16),
                         preferred_element_type=jnp.float32)
        out_ref[...] = shared + acc

        for p in (peers[0], peers[2], peers[1]):
            recv = pltpu.make_async_remote_copy(
                src_ref=comm_ref.at[p],
                dst_ref=comm_ref.at[p],
                send_sem=send_sems.at[0],
                recv_sem=recv_sems.at[p],
                device_id=(p,),
                device_id_type=pl.DeviceIdType.MESH,
            )
            recv.wait_recv()
            out_ref[...] = out_ref[...] + comm_ref[p].astype(jnp.float32)

        for rdma in sends:
            rdma.wait_send()

    return pl.pallas_call(
        body,
        out_shape=jax.ShapeDtypeStruct((n_tok, d_ff), jnp.float32),
        in_specs=[pl.BlockSpec(memory_space=pltpu.VMEM)] * 5,
        out_specs=pl.BlockSpec(memory_space=pltpu.VMEM),
        scratch_shapes=[
            pltpu.VMEM((N_DEV, n_tok, d_ff), jnp.bfloat16),
            pltpu.SemaphoreType.DMA((N_DEV - 1,)),
            pltpu.SemaphoreType.DMA((N_DEV,)),
        ],
        compiler_params=pltpu.CompilerParams(collective_id=0),
    )(x, router_W, route_idx, expert_W, shared_W)
